# baseline (speedup 1.0000x reference)
import functools

import jax
import jax.numpy as jnp
from jax import lax
from jax.experimental import pallas as pl
from jax.experimental.pallas import tpu as pltpu
from jax.experimental.pallas import tpu_sc as plsc

N = 10000
E = 160000
D_IN = 256
D_H = 512

NC = 2
NS = 16
CW = 128

NPAD = 10240
NPT = NPAD // NS
EPAD = 163840
EPT = EPAD // NS
WPT = EPT // 128
GRP = 40
NG = WPT // GRP


def _seg_sum_body(nc, with_deg, *refs):
    src_hbm, dst_hbm = refs[0], refs[1]
    tabs = refs[2:2 + nc]
    pos = 2 + nc
    outs = refs[pos:pos + nc]
    pos += nc
    if with_deg:
        dego = refs[pos]
        pos += 1
    (srcv, dstv, rows_a, rows_b, onesv, z1v, acc, dacc,
     semg_a, semg_b, sems_a, sems_b) = refs[pos:]
    bufs = (rows_a, rows_b)
    semg = (semg_a, semg_b)
    sems = (sems_a, sems_b)

    cid = lax.axis_index("c")
    sid = lax.axis_index("s")

    zeros16 = jnp.zeros((16,), jnp.float32)
    if with_deg:
        ones16 = jnp.ones((16,), jnp.float32)
        for i in range(8):
            onesv[pl.ds(i * 16, 16)] = ones16
            z1v[pl.ds(i * 16, 16)] = zeros16

    for k in range(nc):
        @pl.when(cid == (k % NC))
        def _(k=k):
            deg_here = with_deg and k == 0
            tab = tabs[k]

            def _zrow(r, c):
                for j in range(CW // 16):
                    rows_a[r, pl.ds(j * 16, 16)] = zeros16
                return c
            lax.fori_loop(0, 128, _zrow, 0)
            for j in range(NPT // 128):
                pltpu.sync_copy(rows_a, acc.at[pl.ds(sid * NPT + j * 128, 128)])
            if deg_here:
                for j in range(NPT // 128):
                    pltpu.sync_copy(z1v, dacc.at[pl.ds(sid * NPT + j * 128, 128)])
            plsc.subcore_barrier()

            def grp(g, c):
                base = sid * WPT + g * GRP
                pltpu.sync_copy(src_hbm.at[pl.ds(base, GRP)], srcv)
                pltpu.sync_copy(dst_hbm.at[pl.ds(base, GRP)], dstv)
                for w in range(GRP):
                    i = w % 2
                    buf = bufs[i]
                    if w >= 2:
                        pltpu.make_async_copy(
                            buf, acc.at[dstv.at[w - 2]], sems[i]).wait()
                    pltpu.async_copy(tab.at[srcv.at[w]], buf, semg[i])
                    if w >= 1:
                        j = (w - 1) % 2
                        pb = bufs[j]
                        pltpu.make_async_copy(
                            tab.at[srcv.at[w - 1]], pb, semg[j]).wait()
                        pltpu.async_copy(
                            pb, acc.at[dstv.at[w - 1]], sems[j], add=True)
                        if deg_here:
                            pltpu.sync_copy(
                                onesv, dacc.at[dstv.at[w - 1]], add=True)
                li = (GRP - 1) % 2
                last = bufs[li]
                pltpu.make_async_copy(
                    tab.at[srcv.at[GRP - 1]], last, semg[li]).wait()
                pltpu.async_copy(
                    last, acc.at[dstv.at[GRP - 1]], sems[li], add=True)
                if deg_here:
                    pltpu.sync_copy(onesv, dacc.at[dstv.at[GRP - 1]], add=True)
                pltpu.make_async_copy(
                    bufs[(GRP - 2) % 2], acc.at[dstv.at[GRP - 2]],
                    sems[(GRP - 2) % 2]).wait()
                pltpu.make_async_copy(
                    last, acc.at[dstv.at[GRP - 1]], sems[li]).wait()
                return c
            lax.fori_loop(0, NG, grp, 0)
            plsc.subcore_barrier()

            pltpu.sync_copy(acc.at[pl.ds(sid * NPT, NPT)],
                            outs[k].at[pl.ds(sid * NPT, NPT)])
            if deg_here:
                pltpu.sync_copy(dacc.at[pl.ds(sid * NPT, NPT)],
                                dego.at[pl.ds(sid * NPT, NPT)])


def _make_seg_sum(nc, with_deg):
    out_type = [jax.ShapeDtypeStruct((NPAD, CW), jnp.float32) for _ in range(nc)]
    if with_deg:
        out_type.append(jax.ShapeDtypeStruct((NPAD,), jnp.float32))
    scratch = [
        pltpu.VMEM((GRP, 128), jnp.int32),
        pltpu.VMEM((GRP, 128), jnp.int32),
        pltpu.VMEM((128, CW), jnp.float32),
        pltpu.VMEM((128, CW), jnp.float32),
        pltpu.VMEM((128,), jnp.float32),
        pltpu.VMEM((128,), jnp.float32),
        pltpu.VMEM_SHARED((NPAD, CW), jnp.float32),
        pltpu.VMEM_SHARED((NPAD,), jnp.float32),
        pltpu.SemaphoreType.DMA,
        pltpu.SemaphoreType.DMA,
        pltpu.SemaphoreType.DMA,
        pltpu.SemaphoreType.DMA,
    ]
    mesh = plsc.VectorSubcoreMesh(core_axis_name="c", subcore_axis_name="s",
                                  num_cores=NC, num_subcores=NS)
    return pl.kernel(functools.partial(_seg_sum_body, nc, with_deg),
                     out_type=out_type, mesh=mesh, scratch_types=scratch)


def _self_body(nc_in, *refs):
    h = [refs[i][...] for i in range(nc_in)]
    wr = refs[nc_in][...]
    b = refs[nc_in + 1][...]
    out = refs[nc_in + 2]
    hmat = jnp.concatenate(h, axis=1)
    out[...] = lax.dot_general(hmat, wr, (((1,), (1,)), ((), ())),
                               preferred_element_type=jnp.float32) + b


def _make_self(nc_in, br=2000):
    k_dim = nc_in * CW
    grid = (N // br,)
    in_specs = [pl.BlockSpec((br, CW), lambda i: (i, 0))] * nc_in
    in_specs += [pl.BlockSpec((D_H, k_dim), lambda i: (0, 0)),
                 pl.BlockSpec((1, D_H), lambda i: (0, 0))]
    out_shape = jax.ShapeDtypeStruct((N, D_H), jnp.float32)
    out_specs = pl.BlockSpec((br, D_H), lambda i: (i, 0))
    return pl.pallas_call(
        functools.partial(_self_body, nc_in),
        grid=grid, in_specs=in_specs, out_specs=out_specs, out_shape=out_shape)


def _comb_body(nc_in, relu, deg_ref, *refs):
    a = [refs[i][...] for i in range(nc_in)]
    s = refs[nc_in][...]
    wl = refs[nc_in + 1][...]
    outs = refs[nc_in + 2:]
    inv = 1.0 / jnp.maximum(deg_ref[...], 1.0)
    amat = (jnp.concatenate(a, axis=1) * inv).astype(jnp.bfloat16)
    acc = lax.dot_general(amat, wl, (((1,), (1,)), ((), ())),
                          preferred_element_type=jnp.float32) + s
    if relu:
        acc = jnp.maximum(acc, 0.0)
    if len(outs) == 1:
        outs[0][...] = acc
    else:
        for j, o in enumerate(outs):
            o[...] = acc[:, j * 128:(j + 1) * 128]


def _comb_half_body(nc_in, relu, deg_ref, *refs):
    a = [refs[i][...] for i in range(nc_in)]
    s = refs[nc_in][...]
    wl = refs[nc_in + 1][...]
    outs = refs[nc_in + 2:]
    inv = 1.0 / jnp.maximum(deg_ref[...], 1.0)
    amat = (jnp.concatenate(a, axis=1) * inv).astype(jnp.bfloat16)
    acc = lax.dot_general(amat, wl, (((1,), (1,)), ((), ())),
                          preferred_element_type=jnp.float32) + s
    if relu:
        acc = jnp.maximum(acc, 0.0)
    for j, o in enumerate(outs):
        o[...] = acc[:, j * 128:(j + 1) * 128]


def _make_comb_half(nc_in, relu, half, br=2000):
    k_dim = nc_in * CW
    hw = D_H // 2
    grid = (N // br,)
    in_specs = [pl.BlockSpec((br, 1), lambda i: (i, 0))]
    in_specs += [pl.BlockSpec((br, CW), lambda i: (i, 0))] * nc_in
    in_specs += [pl.BlockSpec((br, hw), lambda i, h=half: (i, h)),
                 pl.BlockSpec((hw, k_dim), lambda i: (0, 0))]
    out_shape = [jax.ShapeDtypeStruct((N, CW), jnp.float32)
                 for _ in range(hw // CW)]
    out_specs = [pl.BlockSpec((br, CW), lambda i: (i, 0))] * (hw // CW)
    return pl.pallas_call(
        functools.partial(_comb_half_body, nc_in, relu),
        grid=grid, in_specs=in_specs, out_specs=out_specs, out_shape=out_shape)


def _make_comb(nc_in, relu, split_out, br=2000):
    k_dim = nc_in * CW
    grid = (N // br,)
    in_specs = [pl.BlockSpec((br, 1), lambda i: (i, 0))]
    in_specs += [pl.BlockSpec((br, CW), lambda i: (i, 0))] * nc_in
    in_specs += [pl.BlockSpec((br, D_H), lambda i: (i, 0)),
                 pl.BlockSpec((D_H, k_dim), lambda i: (0, 0))]
    if split_out:
        out_shape = [jax.ShapeDtypeStruct((N, CW), jnp.float32)
                     for _ in range(D_H // CW)]
        out_specs = [pl.BlockSpec((br, CW), lambda i: (i, 0))] * (D_H // CW)
    else:
        out_shape = jax.ShapeDtypeStruct((N, D_H), jnp.float32)
        out_specs = pl.BlockSpec((br, D_H), lambda i: (i, 0))
    return pl.pallas_call(
        functools.partial(_comb_body, nc_in, relu),
        grid=grid, in_specs=in_specs, out_specs=out_specs, out_shape=out_shape)


def kernel(x, edge_index, W1_l, W1_r, b1, W2_l, W2_r, b2, W3_l, W3_r, b3):
    src = edge_index[0].astype(jnp.int32)
    dst = edge_index[1].astype(jnp.int32)
    pad = EPAD - E
    padi = jnp.arange(pad, dtype=jnp.int32)
    src_p = jnp.concatenate([src, (padi * 53) & 8191])
    dst_p = jnp.concatenate([dst, N + (padi & 127)])
    src2 = src_p.reshape(EPAD // 128, 128)
    dst2 = dst_p.reshape(EPAD // 128, 128)

    x0 = x[:, :CW]
    x1 = x[:, CW:]

    w1l = W1_l.astype(jnp.bfloat16)
    w2l = W2_l.astype(jnp.bfloat16)
    hw = D_H // 2

    a0, a1, deg = _make_seg_sum(2, True)(src2, dst2, x0, x1)
    s1 = _make_self(2)(x0, x1, W1_r, b1.reshape(1, -1))
    deg2 = deg.reshape(NPAD, 1)
    h1a = _make_comb_half(2, True, 0)(deg2, a0, a1, s1, w1l[:hw])
    h1b = _make_comb_half(2, True, 1)(deg2, a0, a1, s1, w1l[hw:])
    g2a = _make_seg_sum(2, False)(src2, dst2, *h1a)
    g2b = _make_seg_sum(2, False)(src2, dst2, *h1b)
    s2 = _make_self(4)(*h1a, *h1b, W2_r, b2.reshape(1, -1))
    h2a = _make_comb_half(4, True, 0)(deg2, *g2a, *g2b, s2, w2l[:hw])
    h2b = _make_comb_half(4, True, 1)(deg2, *g2a, *g2b, s2, w2l[hw:])
    g3a = _make_seg_sum(2, False)(src2, dst2, *h2a)
    g3b = _make_seg_sum(2, False)(src2, dst2, *h2b)
    s3 = _make_self(4)(*h2a, *h2b, W3_r, b3.reshape(1, -1))
    out = _make_comb(4, False, False)(deg2, *g3a, *g3b, s3,
                                      W3_l.astype(jnp.bfloat16))
    return out


# speedup vs baseline: 1.0121x; 1.0121x over previous
import functools

import jax
import jax.numpy as jnp
from jax import lax
from jax.experimental import pallas as pl
from jax.experimental.pallas import tpu as pltpu
from jax.experimental.pallas import tpu_sc as plsc

N = 10000
E = 160000
D_IN = 256
D_H = 512

NC = 2
NS = 16
CW = 128

NPAD = 10240
NPT = NPAD // NS
EPAD = 163840
EPT = EPAD // NS
WPT = EPT // 128
GRP = 40
NG = WPT // GRP


def _seg_sum_body(nc, with_deg, *refs):
    src_hbm, dst_hbm = refs[0], refs[1]
    tabs = refs[2:2 + nc]
    pos = 2 + nc
    outs = refs[pos:pos + nc]
    pos += nc
    if with_deg:
        dego = refs[pos]
        pos += 1
    (srcv, dstv, rows_a, rows_b, onesv, z1v, acc, dacc,
     semg_a, semg_b, sems_a, sems_b) = refs[pos:]
    bufs = (rows_a, rows_b)
    semg = (semg_a, semg_b)
    sems = (sems_a, sems_b)

    cid = lax.axis_index("c")
    sid = lax.axis_index("s")

    zeros16 = jnp.zeros((16,), jnp.float32)
    if with_deg:
        ones16 = jnp.ones((16,), jnp.float32)
        for i in range(8):
            onesv[pl.ds(i * 16, 16)] = ones16
            z1v[pl.ds(i * 16, 16)] = zeros16

    for k in range(nc):
        @pl.when(cid == (k % NC))
        def _(k=k):
            deg_here = with_deg and k == 0
            tab = tabs[k]

            def _zrow(r, c):
                for j in range(CW // 16):
                    rows_a[r, pl.ds(j * 16, 16)] = zeros16
                return c
            lax.fori_loop(0, 128, _zrow, 0)
            for j in range(NPT // 128):
                pltpu.sync_copy(rows_a, acc.at[pl.ds(sid * NPT + j * 128, 128)])
            if deg_here:
                for j in range(NPT // 128):
                    pltpu.sync_copy(z1v, dacc.at[pl.ds(sid * NPT + j * 128, 128)])
            plsc.subcore_barrier()

            def grp(g, c):
                base = sid * WPT + g * GRP
                pltpu.sync_copy(src_hbm.at[pl.ds(base, GRP)], srcv)
                pltpu.sync_copy(dst_hbm.at[pl.ds(base, GRP)], dstv)
                for w in range(GRP):
                    i = w % 2
                    buf = bufs[i]
                    if w >= 2:
                        pltpu.make_async_copy(
                            buf, acc.at[dstv.at[w - 2]], sems[i]).wait()
                    pltpu.async_copy(tab.at[srcv.at[w]], buf, semg[i])
                    if w >= 1:
                        j = (w - 1) % 2
                        pb = bufs[j]
                        pltpu.make_async_copy(
                            tab.at[srcv.at[w - 1]], pb, semg[j]).wait()
                        pltpu.async_copy(
                            pb, acc.at[dstv.at[w - 1]], sems[j], add=True)
                        if deg_here:
                            pltpu.sync_copy(
                                onesv, dacc.at[dstv.at[w - 1]], add=True)
                li = (GRP - 1) % 2
                last = bufs[li]
                pltpu.make_async_copy(
                    tab.at[srcv.at[GRP - 1]], last, semg[li]).wait()
                pltpu.async_copy(
                    last, acc.at[dstv.at[GRP - 1]], sems[li], add=True)
                if deg_here:
                    pltpu.sync_copy(onesv, dacc.at[dstv.at[GRP - 1]], add=True)
                pltpu.make_async_copy(
                    bufs[(GRP - 2) % 2], acc.at[dstv.at[GRP - 2]],
                    sems[(GRP - 2) % 2]).wait()
                pltpu.make_async_copy(
                    last, acc.at[dstv.at[GRP - 1]], sems[li]).wait()
                return c
            lax.fori_loop(0, NG, grp, 0)
            plsc.subcore_barrier()

            pltpu.sync_copy(acc.at[pl.ds(sid * NPT, NPT)],
                            outs[k].at[pl.ds(sid * NPT, NPT)])
            if deg_here:
                pltpu.sync_copy(dacc.at[pl.ds(sid * NPT, NPT)],
                                dego.at[pl.ds(sid * NPT, NPT)])


def _make_seg_sum(nc, with_deg):
    out_type = [jax.ShapeDtypeStruct((NPAD, CW), jnp.float32) for _ in range(nc)]
    if with_deg:
        out_type.append(jax.ShapeDtypeStruct((NPAD,), jnp.float32))
    scratch = [
        pltpu.VMEM((GRP, 128), jnp.int32),
        pltpu.VMEM((GRP, 128), jnp.int32),
        pltpu.VMEM((128, CW), jnp.float32),
        pltpu.VMEM((128, CW), jnp.float32),
        pltpu.VMEM((128,), jnp.float32),
        pltpu.VMEM((128,), jnp.float32),
        pltpu.VMEM_SHARED((NPAD, CW), jnp.float32),
        pltpu.VMEM_SHARED((NPAD,), jnp.float32),
        pltpu.SemaphoreType.DMA,
        pltpu.SemaphoreType.DMA,
        pltpu.SemaphoreType.DMA,
        pltpu.SemaphoreType.DMA,
    ]
    mesh = plsc.VectorSubcoreMesh(core_axis_name="c", subcore_axis_name="s",
                                  num_cores=NC, num_subcores=NS)
    return pl.kernel(functools.partial(_seg_sum_body, nc, with_deg),
                     out_type=out_type, mesh=mesh, scratch_types=scratch)


def _self_body(nc_in, *refs):
    h = [refs[i][...] for i in range(nc_in)]
    wr = refs[nc_in][...]
    b = refs[nc_in + 1][...]
    out = refs[nc_in + 2]
    hmat = jnp.concatenate(h, axis=1)
    out[...] = lax.dot_general(hmat, wr, (((1,), (1,)), ((), ())),
                               preferred_element_type=jnp.float32) + b


def _make_self(nc_in, br=2000):
    k_dim = nc_in * CW
    grid = (N // br,)
    in_specs = [pl.BlockSpec((br, CW), lambda i: (i, 0))] * nc_in
    in_specs += [pl.BlockSpec((D_H, k_dim), lambda i: (0, 0)),
                 pl.BlockSpec((1, D_H), lambda i: (0, 0))]
    out_shape = jax.ShapeDtypeStruct((N, D_H), jnp.float32)
    out_specs = pl.BlockSpec((br, D_H), lambda i: (i, 0))
    return pl.pallas_call(
        functools.partial(_self_body, nc_in),
        grid=grid, in_specs=in_specs, out_specs=out_specs, out_shape=out_shape)


def _comb_body(nc_in, relu, deg_ref, *refs):
    a = [refs[i][...] for i in range(nc_in)]
    s = refs[nc_in][...]
    wl = refs[nc_in + 1][...]
    outs = refs[nc_in + 2:]
    inv = 1.0 / jnp.maximum(deg_ref[...], 1.0)
    amat = (jnp.concatenate(a, axis=1) * inv).astype(jnp.bfloat16)
    acc = lax.dot_general(amat, wl, (((1,), (1,)), ((), ())),
                          preferred_element_type=jnp.float32) + s
    if relu:
        acc = jnp.maximum(acc, 0.0)
    if len(outs) == 1:
        outs[0][...] = acc
    else:
        for j, o in enumerate(outs):
            o[...] = acc[:, j * 128:(j + 1) * 128]


def _make_comb(nc_in, relu, split_out, br=2000):
    k_dim = nc_in * CW
    grid = (N // br,)
    in_specs = [pl.BlockSpec((br, 1), lambda i: (i, 0))]
    in_specs += [pl.BlockSpec((br, CW), lambda i: (i, 0))] * nc_in
    in_specs += [pl.BlockSpec((br, D_H), lambda i: (i, 0)),
                 pl.BlockSpec((D_H, k_dim), lambda i: (0, 0))]
    if split_out:
        out_shape = [jax.ShapeDtypeStruct((N, CW), jnp.float32)
                     for _ in range(D_H // CW)]
        out_specs = [pl.BlockSpec((br, CW), lambda i: (i, 0))] * (D_H // CW)
    else:
        out_shape = jax.ShapeDtypeStruct((N, D_H), jnp.float32)
        out_specs = pl.BlockSpec((br, D_H), lambda i: (i, 0))
    return pl.pallas_call(
        functools.partial(_comb_body, nc_in, relu),
        grid=grid, in_specs=in_specs, out_specs=out_specs, out_shape=out_shape)


def kernel(x, edge_index, W1_l, W1_r, b1, W2_l, W2_r, b2, W3_l, W3_r, b3):
    src = edge_index[0].astype(jnp.int32)
    dst = edge_index[1].astype(jnp.int32)
    pad = EPAD - E
    padi = jnp.arange(pad, dtype=jnp.int32)
    src_p = jnp.concatenate([src, (padi * 53) & 8191])
    dst_p = jnp.concatenate([dst, N + (padi & 127)])
    src2 = src_p.reshape(EPAD // 128, 128)
    dst2 = dst_p.reshape(EPAD // 128, 128)

    x0 = x[:, :CW]
    x1 = x[:, CW:]

    seg2 = _make_seg_sum(2, True)
    seg4a = _make_seg_sum(4, False)
    seg4b = _make_seg_sum(4, False)

    a0, a1, deg = seg2(src2, dst2, x0, x1)
    s1 = _make_self(2)(x0, x1, W1_r, b1.reshape(1, -1))
    deg2 = deg.reshape(NPAD, 1)
    h1 = _make_comb(2, True, True)(deg2, a0, a1, s1,
                                   W1_l.astype(jnp.bfloat16))
    g2 = seg4a(src2, dst2, *h1)
    s2 = _make_self(4)(*h1, W2_r, b2.reshape(1, -1))
    h2 = _make_comb(4, True, True)(deg2, *g2, s2,
                                   W2_l.astype(jnp.bfloat16))
    g3 = seg4b(src2, dst2, *h2)
    s3 = _make_self(4)(*h2, W3_r, b3.reshape(1, -1))
    out = _make_comb(4, False, False)(deg2, *g3, s3,
                                      W3_l.astype(jnp.bfloat16))
    return out
